# baseline (speedup 1.0000x reference)
import functools

import jax
import jax.numpy as jnp
from jax import lax
from jax.experimental import pallas as pl
from jax.experimental.pallas import tpu as pltpu
from jax.experimental.pallas import tpu_sc as plsc

N = 10000
E = 80000
F = 128
H = 32
H2 = 64
D = 64
STEPS = 6

NC = 2
NS = 16
NW = NC * NS
CH = 128
CPW = 20
EPAD = NW * CPW * CH
NPAD = 10112
RPT = NPAD // NS
BE = 512
ECH = E // CH

_f32 = jnp.float32
_bf16 = jnp.bfloat16
_VMEM100 = pltpu.CompilerParams(vmem_limit_bytes=100 * 1024 * 1024)


def _leaky(t):
    return jnp.where(t > 0, t, 0.1 * t)


def _dotbf(a, b):
    return jax.lax.dot_general(
        a.astype(_bf16), b.astype(_bf16), (((a.ndim - 1,), (0,)), ((), ())),
        preferred_element_type=_f32)


def _b16(t):
    return t.astype(_bf16).astype(_f32)


def _pro_body(x_ref, ef_ref, pw_ref, pb_ref, w2_ref, b2_ref,
              h0_ref, nl_ref, ne_ref):
    xv = x_ref[...]
    mu = jnp.mean(xv, axis=0, keepdims=True)
    var = jnp.mean((xv - mu) ** 2, axis=0, keepdims=True)
    nx = (xv - mu) / (jnp.sqrt(var) + 1e-6)
    h0_ref[...] = jnp.maximum(_dotbf(nx, pw_ref[...]) + pb_ref[...], 0.0)
    nl_ref[...] = _leaky(_dotbf(nx, w2_ref[...]) + b2_ref[...])
    ef = ef_ref[...]
    emu = jnp.mean(ef)
    evar = jnp.mean((ef - emu) ** 2)
    ne_ref[...] = (ef - emu) / (jnp.sqrt(evar) + 1e-6)


_prologue = pl.pallas_call(
    _pro_body,
    compiler_params=_VMEM100,
    out_shape=[
        jax.ShapeDtypeStruct((N, H), _f32),
        jax.ShapeDtypeStruct((N, H), _f32),
        jax.ShapeDtypeStruct((ECH, CH), _f32),
    ])


_sc_mesh = plsc.VectorSubcoreMesh(core_axis_name="c", subcore_axis_name="s")


@functools.partial(
    pl.kernel,
    out_type=jax.ShapeDtypeStruct((EPAD, H), _f32),
    mesh=_sc_mesh,
    scratch_types=[
        pltpu.VMEM((CPW, CH), jnp.int32),
        pltpu.VMEM((CH, H), _f32),
        pltpu.SemaphoreType.DMA,
    ],
    compiler_params=pltpu.CompilerParams(use_tc_tiling_on_sc=False),
)
def _sc_gather(h_hbm, src_hbm, out_hbm, src_v, gbuf, sem):
    cid = lax.axis_index("c")
    sid = lax.axis_index("s")
    wid = cid * NS + sid
    pltpu.sync_copy(src_hbm.at[wid], src_v)

    def chunk(j, carry):
        pltpu.async_copy(h_hbm.at[src_v.at[j]], gbuf, sem).wait()
        base = (wid * CPW + j) * CH
        pltpu.sync_copy(gbuf, out_hbm.at[pl.ds(base, CH)])
        return carry

    lax.fori_loop(0, CPW, chunk, 0)


def _msg_body(ne_ref, w1_ref, b1e_ref, w2_ref, b2e_ref, hs_ref, msg_ref):
    ne16 = _b16(ne_ref[...])
    w16 = _b16(w1_ref[...])
    a = jnp.maximum(ne16 * w16 + b1e_ref[...], 0.0)
    ew = _dotbf(a, w2_ref[...]) + b2e_ref[...]
    ew16 = _b16(ew)
    hs16 = _b16(hs_ref[...])
    msg = hs16[:, 0:1] * ew16[:, 0:H]
    for i in range(1, H):
        msg = msg + hs16[:, i:i + 1] * ew16[:, i * H:(i + 1) * H]
    b = pl.program_id(0)
    rows = lax.broadcasted_iota(jnp.int32, (BE, 1), 0) + b * BE
    msg_ref[...] = jnp.where(rows < E, msg, 0.0)


_msg = pl.pallas_call(
    _msg_body,
    grid=(EPAD // BE,),
    in_specs=[
        pl.BlockSpec((BE, 1), lambda b: (b, 0)),
        pl.BlockSpec((1, H), lambda b: (0, 0)),
        pl.BlockSpec((1, H), lambda b: (0, 0)),
        pl.BlockSpec((H, H * H), lambda b: (0, 0)),
        pl.BlockSpec((1, H * H), lambda b: (0, 0)),
        pl.BlockSpec((BE, H), lambda b: (b, 0)),
    ],
    out_specs=pl.BlockSpec((BE, H), lambda b: (b, 0)),
    out_shape=jax.ShapeDtypeStruct((EPAD, H), _f32),
    compiler_params=_VMEM100)


@functools.partial(
    pl.kernel,
    out_type=jax.ShapeDtypeStruct((NC, NPAD, H), _f32),
    mesh=_sc_mesh,
    scratch_types=[
        pltpu.VMEM((CPW, CH), jnp.int32),
        pltpu.VMEM((CH, H), _f32),
        pltpu.VMEM_SHARED((NPAD, H), _f32),
        pltpu.SemaphoreType.DMA,
    ],
    compiler_params=pltpu.CompilerParams(use_tc_tiling_on_sc=False),
)
def _sc_scatter(msg_hbm, dst_hbm, z_hbm, out_hbm, dst_v, mbuf, acc, sem):
    cid = lax.axis_index("c")
    sid = lax.axis_index("s")
    wid = cid * NS + sid
    pltpu.sync_copy(z_hbm.at[pl.ds(sid * RPT, RPT)],
                    acc.at[pl.ds(sid * RPT, RPT)])
    pltpu.sync_copy(dst_hbm.at[wid], dst_v)
    plsc.subcore_barrier()

    def chunk(j, carry):
        base = (wid * CPW + j) * CH
        pltpu.sync_copy(msg_hbm.at[pl.ds(base, CH)], mbuf)
        pltpu.sync_copy(mbuf, acc.at[dst_v.at[j]], add=True)
        return carry

    lax.fori_loop(0, CPW, chunk, 0)
    plsc.subcore_barrier()
    pltpu.sync_copy(acc.at[pl.ds(sid * RPT, RPT)],
                    out_hbm.at[cid, pl.ds(sid * RPT, RPT)])


def _gru_body(part_ref, hid_ref, wih_ref, whh_ref, bih_ref, bhh_ref,
              nb_ref, h_ref):
    agg = part_ref[0, :N, :] + part_ref[1, :N, :] + nb_ref[...]
    m = jnp.maximum(agg, 0.0)
    hid = hid_ref[...]
    gi = _dotbf(m, wih_ref[...]) + bih_ref[...]
    gh = _dotbf(hid, whh_ref[...]) + bhh_ref[...]
    r = jax.nn.sigmoid(gi[:, :H] + gh[:, :H])
    z = jax.nn.sigmoid(gi[:, H:2 * H] + gh[:, H:2 * H])
    nn = jnp.tanh(gi[:, 2 * H:] + r * gh[:, 2 * H:])
    h_ref[...] = (1.0 - z) * nn + z * hid


_gru = pl.pallas_call(
    _gru_body,
    out_shape=jax.ShapeDtypeStruct((N, H), _f32),
    compiler_params=_VMEM100)


def _epi_body(h_ref, nlp_ref, hp_ref, mask_ref, dfs_ref, w1_ref, b1_ref,
              w3_ref, b3_ref, w4_ref, b4_ref, out_ref):
    mp = _leaky(h_ref[...])
    maskf = (mask_ref[...] == 1).astype(_f32)
    dsum = _dotbf(maskf, mp)
    mu = jnp.mean(dsum, axis=0, keepdims=True)
    var = jnp.mean((dsum - mu) ** 2, axis=0, keepdims=True)
    dev_sum = (dsum - mu) / (jnp.sqrt(var) + 1e-6)
    dfs = dfs_ref[...]
    dmu = jnp.mean(dfs, axis=0, keepdims=True)
    dvar = jnp.mean((dfs - dmu) ** 2, axis=0, keepdims=True)
    ndf = (dfs - dmu) / (jnp.sqrt(dvar) + 1e-6)
    dev_emb = _leaky(_dotbf(ndf, w1_ref[...]) + b1_ref[...])
    rep_l = jnp.broadcast_to(nlp_ref[...], (D, H))
    rep_e = jnp.broadcast_to(_leaky(hp_ref[...]), (D, H))
    concat = jnp.concatenate([dev_emb, rep_l, rep_e, dev_sum], axis=1)
    hh = _leaky(_dotbf(concat, w3_ref[...]) + b3_ref[...])
    out_ref[...] = _dotbf(hh, w4_ref[...]) + b4_ref[...]


_epilogue = pl.pallas_call(
    _epi_body,
    out_shape=jax.ShapeDtypeStruct((D, 1), _f32),
    compiler_params=_VMEM100)


def kernel(x, edge_feat, device_feat_state, edge_index, device_assign_state,
           pred_node, proj_W, proj_b, eW1, eb1, eW2, eb2, nn_bias, W_ih,
           W_hh, b_ih, b_hh, W1, b1, W2, b2, W3, b3, W4, b4):
    ef2 = edge_feat.reshape(ECH, CH)
    h0, nl, ne2 = _prologue(x, ef2, proj_W, proj_b.reshape(1, H),
                            W2, b2.reshape(1, H))
    pad = EPAD - E
    ne_pad = jnp.pad(ne2.reshape(-1), (0, pad)).reshape(EPAD, 1)
    src3 = jnp.pad(edge_index[0], (0, pad)).reshape(NW, CPW, CH)
    dst3 = jnp.pad(edge_index[1], (0, pad)).reshape(NW, CPW, CH)
    zeros = jnp.zeros((NPAD, H), _f32)
    eb1r = eb1.reshape(1, H)
    eb2r = eb2.reshape(1, H * H)
    bih = b_ih.reshape(1, 3 * H)
    bhh = b_hh.reshape(1, 3 * H)
    nb = nn_bias.reshape(1, H)
    h = h0
    for _ in range(STEPS):
        hs = _sc_gather(h, src3)
        msg = _msg(ne_pad, eW1, eb1r, eW2, eb2r, hs)
        part = _sc_scatter(msg, dst3, zeros)
        h = _gru(part, h, W_ih, W_hh, bih, bhh, nb)
    nlp = lax.dynamic_slice(nl, (pred_node, 0), (1, H))
    hp = lax.dynamic_slice(h, (pred_node, 0), (1, H))
    out = _epilogue(h, nlp, hp, device_assign_state, device_feat_state,
                    W1, b1.reshape(1, H), W3, b3.reshape(1, H2),
                    W4, b4.reshape(1, 1))
    return out.reshape(-1)
